# baseline (speedup 1.0000x reference)
import jax
import jax.numpy as jnp
from jax import lax
from jax.experimental import pallas as pl
from jax.experimental.pallas import tpu as pltpu
from jax.experimental.pallas import tpu_sc as plsc

_R = 2
_INV2SX = 1.0 / (2.0 * 0.05 ** 2)
_INV2SH = 1.0 / (2.0 * 0.1 ** 2)
_SX = _INV2SX ** 0.5
_SH = _INV2SH ** 0.5
_PAD = 1e4

_B, _H, _W = 4, 192, 640
_WP = 656
_HP = _H + 2 * _R
_NW = 32
_L = 16

_SC_B = 3
_BLK = _H // _NW
_PW = _BLK + 2 * _R
_WS = 672
_CO = 16


def _sc_body(xy1_h, dgt_h, msk_h, hsv_h, dpp_h,
             psum_h, cnt_h,
             stage_xy1, stage_hsv, stage_dpp,
             xyzp_s, hsvp_s, xyzg_s, hsvg_s, dgt_v, msk_v,
             psum_st, cnt_st, sem):
    wid = lax.axis_index("s") * 2 + lax.axis_index("c")
    yg = wid * _BLK
    HW = _H * _W
    PWW = _PW * _WS
    SP = _PW * _W

    gt_cps = []
    for c in range(3):
        gt_cps.append(pltpu.async_copy(
            xy1_h.at[pl.ds(c * HW + yg * _W, _BLK * _W)],
            xyzg_s.at[pl.ds(c * (_BLK * _W), _BLK * _W)], sem))
    gt_cps.append(pltpu.async_copy(
        dgt_h.at[pl.ds(yg * _W, _BLK * _W)], dgt_v, sem))
    for c in range(3):
        gt_cps.append(pltpu.async_copy(
            hsv_h.at[pl.ds(c * HW + yg * _W, _BLK * _W)],
            hsvg_s.at[pl.ds(c * (_BLK * _W), _BLK * _W)], sem))
    gt_cps.append(pltpu.async_copy(
        msk_h.at[pl.ds(yg * _W, _BLK * _W)], msk_v, sem))

    padv = jnp.full((_L,), _PAD * _SX, jnp.float32)

    def prefill(j, carry):
        sl = pl.ds(j * _L, _L)
        xyzp_s[sl] = padv
        hsvp_s[sl] = padv
        return carry
    lax.fori_loop(0, 3 * PWW // _L, prefill, 0)

    def loader(slots, ib_off, nrows, r0):
        def fn():
            cps = []
            for c in range(3):
                cps.append(pltpu.async_copy(
                    xy1_h.at[pl.ds(c * HW + r0 * _W, nrows * _W)],
                    stage_xy1.at[pl.ds(c * SP, nrows * _W)], sem))
                cps.append(pltpu.async_copy(
                    hsv_h.at[pl.ds(c * HW + r0 * _W, nrows * _W)],
                    stage_hsv.at[pl.ds(c * SP, nrows * _W)], sem))
            cps.append(pltpu.async_copy(
                dpp_h.at[pl.ds(r0 * _W, nrows * _W)],
                stage_dpp.at[pl.ds(0, nrows * _W)], sem))
            for cp in cps:
                cp.wait()
            for s in slots:
                ib = s - ib_off

                def dist(k, carry):
                    off = k * _L
                    d = stage_dpp[pl.ds(ib * _W + off, _L)] * _SX
                    for c in range(3):
                        dst = pl.ds(c * PWW + s * _WS + _CO + off, _L)
                        ssl = pl.ds(c * SP + ib * _W + off, _L)
                        xyzp_s[dst] = stage_xy1[ssl] * d
                        hsvp_s[dst] = stage_hsv[ssl] * _SH
                    return carry
                lax.fori_loop(0, _W // _L, dist, 0)
        return fn

    pl.when(wid == 0)(loader(range(2, _PW), 2, _PW - 2, yg))
    pl.when(wid == _NW - 1)(loader(range(0, _PW - 2), 0, _PW - 2, yg - 2))
    pl.when(jnp.logical_and(wid > 0, wid < _NW - 1))(
        loader(range(0, _PW), 0, _PW, yg - 2))
    for cp in gt_cps:
        cp.wait()

    def pre_gt(j, carry):
        off = j * _L
        d = dgt_v[pl.ds(off, _L)] * _SX
        for c in range(3):
            sl = pl.ds(c * (_BLK * _W) + off, _L)
            xyzg_s[sl] = xyzg_s[sl] * d
            hsvg_s[sl] = hsvg_s[sl] * _SH
        return carry
    lax.fori_loop(0, _BLK * _W // _L, pre_gt, 0)

    nchunk = _W // _L

    def chunk(j, carry):
        a_acc, a_cnt = carry
        ri = j // nchunk
        xc = j - ri * nchunk
        goff = j * _L
        gsl = pl.ds(goff, _L)
        m = msk_v[gsl]
        gx = [xyzg_s[pl.ds(c * (_BLK * _W) + goff, _L)] for c in range(3)]
        gh = [hsvg_s[pl.ds(c * (_BLK * _W) + goff, _L)] for c in range(3)]
        pbase = ri * _WS + xc * _L + _CO - _R
        a = jnp.zeros((_L,), jnp.float32)
        for dy in range(2 * _R + 1):
            for dx in range(2 * _R + 1):
                nb = pbase + dy * _WS + dx
                t = jnp.zeros((_L,), jnp.float32)
                for c in range(3):
                    d = gx[c] - xyzp_s[pl.ds(c * PWW + nb, _L)]
                    t = t + d * d
                for c in range(3):
                    d = gh[c] - hsvp_s[pl.ds(c * PWW + nb, _L)]
                    t = t + d * d
                a = a + jnp.exp(-t)
        return (a_acc + a * m, a_cnt + m)

    acc, cnt = lax.fori_loop(
        0, _BLK * nchunk, chunk,
        (jnp.zeros((_L,), jnp.float32), jnp.zeros((_L,), jnp.float32)))

    psum_st[...] = acc
    cnt_st[...] = cnt
    pltpu.sync_copy(psum_st, psum_h.at[pl.ds(wid * _L, _L)])
    pltpu.sync_copy(cnt_st, cnt_h.at[pl.ds(wid * _L, _L)])


def _tc_body(xy1_ref, dgt_ref, msk_ref, hsv_ref,
             xy1p_ref, dpp_ref, hsvp_ref,
             psum_ref, cnt_ref):
    H, W = dgt_ref.shape[1], dgt_ref.shape[2]
    xy1 = xy1_ref[0]
    dgt = dgt_ref[0]
    hsv = hsv_ref[0]
    xyzg = xy1 * dgt[None]
    xy1p = xy1p_ref[0]
    dpp = dpp_ref[0]
    xyzp = xy1p * dpp[None]
    hsvp = hsvp_ref[0]
    total = jnp.zeros((H, W), dtype=jnp.float32)
    for dy in range(2 * _R + 1):
        for dx in range(2 * _R + 1):
            xs = xyzp[:, dy:dy + H, dx:dx + W]
            hs = hsvp[:, dy:dy + H, dx:dx + W]
            d2 = jnp.sum((xyzg - xs) ** 2, axis=0)
            h2 = jnp.sum((hsv - hs) ** 2, axis=0)
            total = total + jnp.exp(-(d2 * _INV2SX + h2 * _INV2SH))
    msk = msk_ref[0]
    psum_ref[0, 0, :] = jnp.full((128,), jnp.sum(total * msk), jnp.float32)
    cnt_ref[0, 0, :] = jnp.full((128,), jnp.sum(msk), jnp.float32)


def kernel(depth_pred, depth_gt, mask_gt, xy1_grid, hsv):
    B, _, H, W = depth_pred.shape
    r = _R
    padhw = ((0, 0), (r, r), (r, _WP - _W - r))
    pad3 = ((0, 0), (0, 0), (r, r), (r, _WP - _W - r))
    dgt = depth_gt[:, 0]
    msk = mask_gt[:, 0].astype(jnp.float32)
    dpp = jnp.pad(depth_pred[:_SC_B, 0], padhw, constant_values=_PAD)
    xy1p = jnp.pad(xy1_grid[:_SC_B], pad3, constant_values=1.0)
    hsvp = jnp.pad(hsv[:_SC_B], pad3, constant_values=_PAD)

    mesh = plsc.VectorSubcoreMesh(core_axis_name="c", subcore_axis_name="s")
    sc_fn = pl.kernel(
        _sc_body,
        out_type=[
            jax.ShapeDtypeStruct((_NW * _L,), jnp.float32),
            jax.ShapeDtypeStruct((_NW * _L,), jnp.float32),
        ],
        mesh=mesh,
        scratch_types=[
            pltpu.VMEM((3 * _PW * _W,), jnp.float32),
            pltpu.VMEM((3 * _PW * _W,), jnp.float32),
            pltpu.VMEM((_PW * _W,), jnp.float32),
            pltpu.VMEM((3 * _PW * _WS,), jnp.float32),
            pltpu.VMEM((3 * _PW * _WS,), jnp.float32),
            pltpu.VMEM((3 * _BLK * _W,), jnp.float32),
            pltpu.VMEM((3 * _BLK * _W,), jnp.float32),
            pltpu.VMEM((_BLK * _W,), jnp.float32),
            pltpu.VMEM((_BLK * _W,), jnp.float32),
            pltpu.VMEM((_L,), jnp.float32),
            pltpu.VMEM((_L,), jnp.float32),
            pltpu.SemaphoreType.DMA,
        ],
    )
    psum_sc, cnt_sc = sc_fn(
        xy1_grid[_SC_B].reshape(-1), dgt[_SC_B].reshape(-1),
        msk[_SC_B].reshape(-1), hsv[_SC_B].reshape(-1),
        depth_pred[_SC_B, 0].reshape(-1))

    b3 = lambda b: (b, 0, 0, 0)
    b2 = lambda b: (b, 0, 0)
    psum_tc, cnt_tc = pl.pallas_call(
        _tc_body,
        grid=(_SC_B,),
        in_specs=[
            pl.BlockSpec((1, 3, H, W), b3),
            pl.BlockSpec((1, H, W), b2),
            pl.BlockSpec((1, H, W), b2),
            pl.BlockSpec((1, 3, H, W), b3),
            pl.BlockSpec((1, 3, _HP, _WP), b3),
            pl.BlockSpec((1, _HP, _WP), b2),
            pl.BlockSpec((1, 3, _HP, _WP), b3),
        ],
        out_specs=[
            pl.BlockSpec((1, 1, 128), lambda b: (b, 0, 0)),
            pl.BlockSpec((1, 1, 128), lambda b: (b, 0, 0)),
        ],
        out_shape=[
            jax.ShapeDtypeStruct((_SC_B, 1, 128), jnp.float32),
            jax.ShapeDtypeStruct((_SC_B, 1, 128), jnp.float32),
        ],
    )(xy1_grid, dgt, msk, hsv, xy1p, dpp, hsvp)

    psum = jnp.sum(psum_sc) + jnp.sum(psum_tc[:, 0, 0])
    n_valid = jnp.sum(cnt_sc) + jnp.sum(cnt_tc[:, 0, 0])
    inp = psum / (n_valid * float((2 * _R + 1) ** 2) + 1e-8)
    return 1.0 - inp


# speedup vs baseline: 1.4883x; 1.4883x over previous
import jax
import jax.numpy as jnp
from jax import lax
from jax.experimental import pallas as pl
from jax.experimental.pallas import tpu as pltpu
from jax.experimental.pallas import tpu_sc as plsc

_R = 2
_INV2SX = 1.0 / (2.0 * 0.05 ** 2)
_INV2SH = 1.0 / (2.0 * 0.1 ** 2)
_SX = _INV2SX ** 0.5
_SH = _INV2SH ** 0.5
_PAD = 1e4

_B, _H, _W = 4, 192, 640
_WP = 656
_HP = _H + 2 * _R
_NW = 32
_L = 16

_SC_B = 3
_BLK = _H // _NW
_PW = _BLK + 2 * _R
_WS = 672
_CO = 16


def _sc_body(xy1_h, dgt_h, msk_h, hsv_h, xy1p_h, dpp_h, hsvp_h,
             psum_h, cnt_h,
             xyzp_s, hsvp_s, dpp_v, xyzg_s, hsvg_s, dgt_v, msk_v,
             psum_st, cnt_st, sem):
    wid = lax.axis_index("s") * 2 + lax.axis_index("c")
    yg = wid * _BLK
    PWW = _PW * _WP

    cps = []
    for c in range(3):
        cps.append(pltpu.async_copy(
            xy1p_h.at[pl.ds(c * (_HP * _WP) + yg * _WP, PWW)],
            xyzp_s.at[pl.ds(c * PWW, PWW)], sem))
    cps.append(pltpu.async_copy(
        dpp_h.at[pl.ds(yg * _WP, PWW)], dpp_v, sem))
    for c in range(3):
        cps.append(pltpu.async_copy(
            hsvp_h.at[pl.ds(c * (_HP * _WP) + yg * _WP, PWW)],
            hsvp_s.at[pl.ds(c * PWW, PWW)], sem))
    for c in range(3):
        cps.append(pltpu.async_copy(
            xy1_h.at[pl.ds(c * (_H * _W) + yg * _W, _BLK * _W)],
            xyzg_s.at[pl.ds(c * (_BLK * _W), _BLK * _W)], sem))
    cps.append(pltpu.async_copy(
        dgt_h.at[pl.ds(yg * _W, _BLK * _W)], dgt_v, sem))
    for c in range(3):
        cps.append(pltpu.async_copy(
            hsv_h.at[pl.ds(c * (_H * _W) + yg * _W, _BLK * _W)],
            hsvg_s.at[pl.ds(c * (_BLK * _W), _BLK * _W)], sem))
    cps.append(pltpu.async_copy(
        msk_h.at[pl.ds(yg * _W, _BLK * _W)], msk_v, sem))
    for cp in cps:
        cp.wait()

    def pre_pred(j, carry):
        off = j * _L
        d = dpp_v[pl.ds(off, _L)] * _SX
        for c in range(3):
            sl = pl.ds(c * PWW + off, _L)
            xyzp_s[sl] = xyzp_s[sl] * d
            hsvp_s[sl] = hsvp_s[sl] * _SH
        return carry
    lax.fori_loop(0, PWW // _L, pre_pred, 0)

    def pre_gt(j, carry):
        off = j * _L
        d = dgt_v[pl.ds(off, _L)] * _SX
        for c in range(3):
            sl = pl.ds(c * (_BLK * _W) + off, _L)
            xyzg_s[sl] = xyzg_s[sl] * d
            hsvg_s[sl] = hsvg_s[sl] * _SH
        return carry
    lax.fori_loop(0, _BLK * _W // _L, pre_gt, 0)

    nchunk = _W // _L

    def chunk(j, carry):
        a_acc, a_cnt = carry
        ri = j // nchunk
        xc = j - ri * nchunk
        goff = j * _L
        gsl = pl.ds(goff, _L)
        m = msk_v[gsl]
        gx = [xyzg_s[pl.ds(c * (_BLK * _W) + goff, _L)] for c in range(3)]
        gh = [hsvg_s[pl.ds(c * (_BLK * _W) + goff, _L)] for c in range(3)]
        pbase = ri * _WP + xc * _L
        a = jnp.zeros((_L,), jnp.float32)
        for dy in range(2 * _R + 1):
            for dx in range(2 * _R + 1):
                nb = pbase + dy * _WP + dx
                t = jnp.zeros((_L,), jnp.float32)
                for c in range(3):
                    d = gx[c] - xyzp_s[pl.ds(c * PWW + nb, _L)]
                    t = t + d * d
                for c in range(3):
                    d = gh[c] - hsvp_s[pl.ds(c * PWW + nb, _L)]
                    t = t + d * d
                a = a + jnp.exp(-t)
        return (a_acc + a * m, a_cnt + m)

    acc, cnt = lax.fori_loop(
        0, _BLK * nchunk, chunk,
        (jnp.zeros((_L,), jnp.float32), jnp.zeros((_L,), jnp.float32)))

    psum_st[...] = acc
    cnt_st[...] = cnt
    pltpu.sync_copy(psum_st, psum_h.at[pl.ds(wid * _L, _L)])
    pltpu.sync_copy(cnt_st, cnt_h.at[pl.ds(wid * _L, _L)])


def _tc_body(xy1_ref, dgt_ref, msk_ref, hsv_ref,
             xy1p_ref, dpp_ref, hsvp_ref,
             psum_ref, cnt_ref):
    H, W = dgt_ref.shape[1], dgt_ref.shape[2]
    xy1 = xy1_ref[0]
    dgt = dgt_ref[0]
    hsv = hsv_ref[0]
    xyzg = xy1 * dgt[None]
    xy1p = xy1p_ref[0]
    dpp = dpp_ref[0]
    xyzp = xy1p * dpp[None]
    hsvp = hsvp_ref[0]
    total = jnp.zeros((H, W), dtype=jnp.float32)
    for dy in range(2 * _R + 1):
        for dx in range(2 * _R + 1):
            xs = xyzp[:, dy:dy + H, dx:dx + W]
            hs = hsvp[:, dy:dy + H, dx:dx + W]
            d2 = jnp.sum((xyzg - xs) ** 2, axis=0)
            h2 = jnp.sum((hsv - hs) ** 2, axis=0)
            total = total + jnp.exp(-(d2 * _INV2SX + h2 * _INV2SH))
    msk = msk_ref[0]
    psum_ref[0, 0, :] = jnp.full((128,), jnp.sum(total * msk), jnp.float32)
    cnt_ref[0, 0, :] = jnp.full((128,), jnp.sum(msk), jnp.float32)


def kernel(depth_pred, depth_gt, mask_gt, xy1_grid, hsv):
    B, _, H, W = depth_pred.shape
    r = _R
    padhw = ((0, 0), (r, r), (r, _WP - _W - r))
    pad3 = ((0, 0), (0, 0), (r, r), (r, _WP - _W - r))
    dgt = depth_gt[:, 0]
    msk = mask_gt[:, 0].astype(jnp.float32)
    dpp = jnp.pad(depth_pred[:, 0], padhw, constant_values=_PAD)
    xy1p = jnp.pad(xy1_grid, pad3, constant_values=1.0)
    hsvp = jnp.pad(hsv, pad3, constant_values=_PAD)
    xy1p_s3, dpp_s3, hsvp_s3 = xy1p[_SC_B], dpp[_SC_B], hsvp[_SC_B]
    xy1p, dpp, hsvp = xy1p[:_SC_B], dpp[:_SC_B], hsvp[:_SC_B]

    mesh = plsc.VectorSubcoreMesh(core_axis_name="c", subcore_axis_name="s")
    sc_fn = pl.kernel(
        _sc_body,
        out_type=[
            jax.ShapeDtypeStruct((_NW * _L,), jnp.float32),
            jax.ShapeDtypeStruct((_NW * _L,), jnp.float32),
        ],
        mesh=mesh,
        scratch_types=[
            pltpu.VMEM((3 * _PW * _WP,), jnp.float32),
            pltpu.VMEM((3 * _PW * _WP,), jnp.float32),
            pltpu.VMEM((_PW * _WP,), jnp.float32),
            pltpu.VMEM((3 * _BLK * _W,), jnp.float32),
            pltpu.VMEM((3 * _BLK * _W,), jnp.float32),
            pltpu.VMEM((_BLK * _W,), jnp.float32),
            pltpu.VMEM((_BLK * _W,), jnp.float32),
            pltpu.VMEM((_L,), jnp.float32),
            pltpu.VMEM((_L,), jnp.float32),
            pltpu.SemaphoreType.DMA,
        ],
    )
    psum_sc, cnt_sc = sc_fn(
        xy1_grid[_SC_B].reshape(-1), dgt[_SC_B].reshape(-1),
        msk[_SC_B].reshape(-1), hsv[_SC_B].reshape(-1),
        xy1p_s3.reshape(-1), dpp_s3.reshape(-1), hsvp_s3.reshape(-1))

    b3 = lambda b: (b, 0, 0, 0)
    b2 = lambda b: (b, 0, 0)
    psum_tc, cnt_tc = pl.pallas_call(
        _tc_body,
        grid=(_SC_B,),
        in_specs=[
            pl.BlockSpec((1, 3, H, W), b3),
            pl.BlockSpec((1, H, W), b2),
            pl.BlockSpec((1, H, W), b2),
            pl.BlockSpec((1, 3, H, W), b3),
            pl.BlockSpec((1, 3, _HP, _WP), b3),
            pl.BlockSpec((1, _HP, _WP), b2),
            pl.BlockSpec((1, 3, _HP, _WP), b3),
        ],
        out_specs=[
            pl.BlockSpec((1, 1, 128), lambda b: (b, 0, 0)),
            pl.BlockSpec((1, 1, 128), lambda b: (b, 0, 0)),
        ],
        out_shape=[
            jax.ShapeDtypeStruct((_SC_B, 1, 128), jnp.float32),
            jax.ShapeDtypeStruct((_SC_B, 1, 128), jnp.float32),
        ],
    )(xy1_grid, dgt, msk, hsv, xy1p, dpp, hsvp)

    psum = jnp.sum(psum_sc) + jnp.sum(psum_tc[:, 0, 0])
    n_valid = jnp.sum(cnt_sc) + jnp.sum(cnt_tc[:, 0, 0])
    inp = psum / (n_valid * float((2 * _R + 1) ** 2) + 1e-8)
    return 1.0 - inp
